# baseline (speedup 1.0000x reference)
import functools
import math

import jax
import jax.numpy as jnp
from jax import lax
from jax.experimental import pallas as pl
from jax.experimental.pallas import tpu as pltpu
from jax.experimental.pallas import tpu_sc as plsc

K = 8
C = 3
A = 1024
N = 1000000
NUM_ITEMS = 500000
MIN_LL = float(math.log(1e-4))

L = 16
SHIFT = 13
IR = 1 << SHIFT
NR = (NUM_ITEMS + IR - 1) // IR
LAST_IR = NUM_ITEMS - (NR - 1) * IR
CH = 2000
PAD_ITEM = 1 << 22
NSAMP_PAD = 62528
NI_PAD = NSAMP_PAD * 16
TBR = NSAMP_PAD // 8
NW = 32


def _tc_body(mu_t_ref, rt_ref, items_ref, lut_ref, bnd_ref):
    step = pl.program_id(0)

    @pl.when(step == 0)
    def _():
        emu = jnp.exp(mu_t_ref[...])
        rt = rt_ref[...]
        logits = [rt[:, c:c + 1] + emu[c:c + 1, :] for c in range(C)]
        m = jnp.maximum(jnp.maximum(logits[0], logits[1]), logits[2])
        se = sum(jnp.exp(lc - m) for lc in logits)
        lse = m + jnp.log(se)
        for c in range(C):
            ll = logits[c] - lse
            lut_ref[:, c * K:(c + 1) * K] = jnp.where(ll > MIN_LL, ll, MIN_LL)
        bnd_ref[...] = jnp.zeros_like(bnd_ref)

    blk = items_ref[...]
    col = blk[:, 0:1]
    thr = lax.broadcasted_iota(jnp.int32, (1, 80), 1) * IR
    cnt = jnp.sum((col < thr).astype(jnp.int32), axis=0)
    bnd_ref[0:1, :] = bnd_ref[0:1, :] + cnt[None, :]


_tc_call = pl.pallas_call(
    _tc_body,
    grid=(8,),
    in_specs=[
        pl.BlockSpec((C, K), lambda i: (0, 0)),
        pl.BlockSpec((A, C), lambda i: (0, 0)),
        pl.BlockSpec((TBR, 16), lambda i: (i, 0)),
    ],
    out_specs=[
        pl.BlockSpec((A, C * K), lambda i: (0, 0)),
        pl.BlockSpec((8, 80), lambda i: (0, 0)),
    ],
    out_shape=[
        jax.ShapeDtypeStruct((A, C * K), jnp.float32),
        jax.ShapeDtypeStruct((8, 80), jnp.int32),
    ],
)


def _sc_body(lut_hbm, bnd_hbm, anno_hbm, annot_hbm, items_hbm, conf_hbm,
             w_hbm, tot_hbm, lut_v, acc, items_v, av_v, cv_v, conf_v,
             wstage, bnd_v):
    wid = lax.axis_index("s") * 2 + lax.axis_index("c")
    pltpu.sync_copy(lut_hbm, lut_v)
    pltpu.sync_copy(bnd_hbm, bnd_v)

    iota = lax.iota(jnp.int32, L)
    lane0 = iota == 0
    not15 = iota != (L - 1)
    zf = jnp.zeros((L,), jnp.float32)

    def run_range(r):
        bvec = bnd_v[pl.ds(r, L)]
        b0 = bvec[0]
        b1 = bvec[1]
        lo = r * IR
        wb = L * jnp.maximum(b0 - 1, 0)
        send = L * b1
        c0 = wb // CH
        c1 = (send + CH - 1) // CH

        def zbody(i, _):
            acc[pl.ds(i * L, L)] = zf
            return 0
        lax.fori_loop(0, (K * IR) // L, zbody, 0)

        def chunk_body(ci, carries):
            s = ci * CH
            pltpu.sync_copy(items_hbm.at[pl.ds(s, CH + L)], items_v)
            pltpu.sync_copy(anno_hbm.at[pl.ds(s, CH)], cv_v)
            pltpu.sync_copy(annot_hbm.at[pl.ds(s, CH)], av_v)
            pltpu.sync_copy(conf_hbm.at[pl.ds(s, CH)], conf_v)

            def blk_body(j, carries):
                o = j * L
                it = items_v[pl.ds(o, L)]
                itn = items_v[pl.ds(o + 1, L)]
                av = av_v[pl.ds(o, L)]
                cv = cv_v[pl.ds(o, L)]
                cf = conf_v[pl.ds(o, L)]
                gidx = av * (C * K) + cv * K
                bnd = it != itn
                itr = it - lo
                itnr = itn - lo
                inr1 = bnd & ((it >> SHIFT) == r)
                inr2 = bnd & ((itn >> SHIFT) == r) & not15
                keepm = lane0 & (lax.rev(it, (0,)) == lax.rev(itn, (0,)))
                new_carries = []
                for k in range(K):
                    g = plsc.load_gather(lut_v, [gidx + k])
                    w = g * cf
                    wstage[pl.ds(k * CH + o, L)] = w
                    wc = w + carries[k]
                    p = plsc.cumsum(wc)
                    plsc.addupdate_scatter(acc, [itr + k * IR], p, mask=inr1)
                    plsc.addupdate_scatter(acc, [itnr + k * IR], -p, mask=inr2)
                    new_carries.append(jnp.where(keepm, lax.rev(p, (0,)), zf))
                return tuple(new_carries)

            carries = lax.fori_loop(0, CH // L, blk_body, carries)
            for k in range(K):
                pltpu.sync_copy(wstage.at[pl.ds(k * CH, CH)],
                                w_hbm.at[k, pl.ds(s, CH)])
            return carries

        carries0 = tuple(zf for _ in range(K))
        lax.fori_loop(c0, c1, chunk_body, carries0)

        @pl.when(r < NR - 1)
        def _():
            for k in range(K):
                pltpu.sync_copy(acc.at[pl.ds(k * IR, IR)],
                                tot_hbm.at[k, pl.ds(lo, IR)])

        @pl.when(r == NR - 1)
        def _():
            for k in range(K):
                pltpu.sync_copy(acc.at[pl.ds(k * IR, LAST_IR)],
                                tot_hbm.at[k, pl.ds(lo, LAST_IR)])

    run_range(wid)

    @pl.when(wid + NW < NR)
    def _():
        run_range(wid + NW)


_sc_call = functools.partial(
    pl.kernel,
    out_type=[
        jax.ShapeDtypeStruct((K, N), jnp.float32),
        jax.ShapeDtypeStruct((K, NUM_ITEMS), jnp.float32),
    ],
    mesh=plsc.VectorSubcoreMesh(core_axis_name="c", subcore_axis_name="s"),
    compiler_params=pltpu.CompilerParams(needs_layout_passes=False,
                                         use_tc_tiling_on_sc=False),
    scratch_types=[
        pltpu.VMEM((A * C * K,), jnp.float32),
        pltpu.VMEM((K * IR,), jnp.float32),
        pltpu.VMEM((CH + L,), jnp.int32),
        pltpu.VMEM((CH,), jnp.int32),
        pltpu.VMEM((CH,), jnp.int32),
        pltpu.VMEM((CH,), jnp.float32),
        pltpu.VMEM((K * CH,), jnp.float32),
        pltpu.VMEM((80,), jnp.int32),
    ],
)(_sc_body)


def kernel(mu, random_table, anno, items, annotators, confidence):
    anno_i = anno.astype(jnp.int32)
    annot_i = annotators.astype(jnp.int32)
    items_p = jnp.pad(items.astype(jnp.int32), (0, NI_PAD - N),
                      constant_values=PAD_ITEM)
    items_tc = items_p.reshape(NSAMP_PAD, 16)

    lut2d, bnd2d = _tc_call(mu.T, random_table, items_tc)
    lut = lut2d.reshape(-1)
    bnd = bnd2d[0]

    weighted, total_ll = _sc_call(lut, bnd, anno_i, annot_i, items_p,
                                  confidence)
    return (weighted, total_ll)


# speedup vs baseline: 2.2286x; 2.2286x over previous
import functools
import math

import jax
import jax.numpy as jnp
from jax import lax
from jax.experimental import pallas as pl
from jax.experimental.pallas import tpu as pltpu
from jax.experimental.pallas import tpu_sc as plsc

K = 8
C = 3
A = 1024
N = 1000000
NUM_ITEMS = 500000
MIN_LL = float(math.log(1e-4))

L = 16
SHIFT = 12
IR = 1 << SHIFT
NR = (NUM_ITEMS + IR - 1) // IR
NT2 = NR * IR
CH = 1024
NSAMP_PAD = 62528
N2 = NSAMP_PAD * 16
NCH = N2 // CH
PAD_ITEM = 1 << 22
TBR = NSAMP_PAD // 8
NW = 32
NB = 144


def _tc_body(mu_t_ref, rt_ref, items_ref, lut_ref, bnd_ref):
    step = pl.program_id(0)

    @pl.when(step == 0)
    def _():
        emu = jnp.exp(mu_t_ref[...])
        rt = rt_ref[...]
        logits = [rt[:, c:c + 1] + emu[c:c + 1, :] for c in range(C)]
        m = jnp.maximum(jnp.maximum(logits[0], logits[1]), logits[2])
        se = sum(jnp.exp(lc - m) for lc in logits)
        lse = m + jnp.log(se)
        for c in range(C):
            ll = logits[c] - lse
            lut_ref[:, c * K:(c + 1) * K] = jnp.where(ll > MIN_LL, ll, MIN_LL)
        bnd_ref[...] = jnp.zeros_like(bnd_ref)

    blk = items_ref[...]
    col = blk[:, 0:1]
    thr = lax.broadcasted_iota(jnp.int32, (1, NB), 1) * IR
    cnt = jnp.sum((col < thr).astype(jnp.int32), axis=0)
    bnd_ref[0:1, :] = bnd_ref[0:1, :] + cnt[None, :]


_tc_call = pl.pallas_call(
    _tc_body,
    grid=(8,),
    in_specs=[
        pl.BlockSpec((C, K), lambda i: (0, 0)),
        pl.BlockSpec((A, C), lambda i: (0, 0)),
        pl.BlockSpec((TBR, 16), lambda i: (i, 0)),
    ],
    out_specs=[
        pl.BlockSpec((A, C * K), lambda i: (0, 0)),
        pl.BlockSpec((8, NB), lambda i: (0, 0)),
    ],
    out_shape=[
        jax.ShapeDtypeStruct((A, C * K), jnp.float32),
        jax.ShapeDtypeStruct((8, NB), jnp.int32),
    ],
)


def _sc_body(lut_hbm, bnd_hbm, anno_hbm, annot_hbm, items_hbm, conf_hbm,
             w_hbm, tot_hbm, lut_v, acc, acc2, items_v, av_v, cv_v, conf_v,
             wstage, bnd_v):
    wid = lax.axis_index("s") * 2 + lax.axis_index("c")
    pltpu.sync_copy(lut_hbm, lut_v)
    pltpu.sync_copy(bnd_hbm, bnd_v)

    iota = lax.iota(jnp.int32, L)
    lane0 = iota == 0
    not15 = iota != (L - 1)
    zf = jnp.zeros((L,), jnp.float32)

    def run_range(r):
        bvec = bnd_v[pl.ds(r, L)]
        b0 = bvec[0]
        b1 = bvec[1]
        lo = r * IR
        wb = L * jnp.maximum(b0 - 1, 0)
        send = L * b1
        c0 = wb // CH
        c1 = (send + CH - 1) // CH

        def zbody(i, _):
            acc[pl.ds(i * L, L)] = zf
            return 0
        lax.fori_loop(0, (K * IR) // L, zbody, 0)

        def chunk_body(ci, carries):
            s = ci * CH
            pltpu.sync_copy(items_hbm.at[pl.ds(s, CH + L)], items_v)
            pltpu.sync_copy(anno_hbm.at[pl.ds(s, CH)], cv_v)
            pltpu.sync_copy(annot_hbm.at[pl.ds(s, CH)], av_v)
            pltpu.sync_copy(conf_hbm.at[pl.ds(s, CH)], conf_v)

            def blk_body(j, carries):
                o = j * L
                it = items_v[pl.ds(o, L)]
                itn = items_v[pl.ds(o + 1, L)]
                av = av_v[pl.ds(o, L)]
                cv = cv_v[pl.ds(o, L)]
                cf = conf_v[pl.ds(o, L)]
                gidx = av * (C * K) + cv * K
                bnd = it != itn
                itr = it - lo
                itnr = itn - lo
                inr1 = bnd & ((it >> SHIFT) == r)
                inr2 = bnd & ((itn >> SHIFT) == r) & not15
                keepm = lane0 & (lax.rev(it, (0,)) == lax.rev(itn, (0,)))
                new_carries = []
                for k in range(K):
                    g = plsc.load_gather(lut_v, [gidx + k])
                    w = g * cf
                    wstage[k, pl.ds(o, L)] = w
                    wc = w + carries[k]
                    p = plsc.cumsum(wc)
                    plsc.addupdate_scatter(acc, [itr + k * IR], p, mask=inr1)
                    plsc.addupdate_scatter(acc, [itnr + k * IR], -p, mask=inr2)
                    new_carries.append(jnp.where(keepm, lax.rev(p, (0,)), zf))
                return tuple(new_carries)

            carries = lax.fori_loop(0, CH // L, blk_body, carries)
            pltpu.sync_copy(wstage, w_hbm.at[:, pl.ds(s, CH)])
            return carries

        carries0 = tuple(zf for _ in range(K))
        lax.fori_loop(c0, c1, chunk_body, carries0)

        def sbody(i, _):
            cc = i * L
            for k in range(K):
                acc2[k, pl.ds(cc, L)] = acc[pl.ds(k * IR + cc, L)]
            return 0
        lax.fori_loop(0, IR // L, sbody, 0)
        pltpu.sync_copy(acc2, tot_hbm.at[:, pl.ds(lo, IR)])

    def maybe_run(r):
        @pl.when(r < NR)
        def _():
            run_range(r)

    maybe_run(wid)
    maybe_run(wid + NW)
    maybe_run(wid + 2 * NW)
    maybe_run(wid + 3 * NW)


_sc_call = functools.partial(
    pl.kernel,
    out_type=[
        jax.ShapeDtypeStruct((K, N2), jnp.float32),
        jax.ShapeDtypeStruct((K, NT2), jnp.float32),
    ],
    mesh=plsc.VectorSubcoreMesh(core_axis_name="c", subcore_axis_name="s"),
    compiler_params=pltpu.CompilerParams(needs_layout_passes=False),
    scratch_types=[
        pltpu.VMEM((A * C * K,), jnp.float32),
        pltpu.VMEM((K * IR,), jnp.float32),
        pltpu.VMEM((K, IR), jnp.float32),
        pltpu.VMEM((CH + L,), jnp.int32),
        pltpu.VMEM((CH,), jnp.int32),
        pltpu.VMEM((CH,), jnp.int32),
        pltpu.VMEM((CH,), jnp.float32),
        pltpu.VMEM((K, CH), jnp.float32),
        pltpu.VMEM((NB,), jnp.int32),
    ],
)(_sc_body)


def kernel(mu, random_table, anno, items, annotators, confidence):
    pad = N2 - N
    anno_p = jnp.pad(anno.astype(jnp.int32), (0, pad))
    annot_p = jnp.pad(annotators.astype(jnp.int32), (0, pad))
    conf_p = jnp.pad(confidence, (0, pad))
    items_p = jnp.pad(items.astype(jnp.int32), (0, pad + L),
                      constant_values=PAD_ITEM)
    items_tc = items_p[:N2].reshape(NSAMP_PAD, 16)

    lut2d, bnd2d = _tc_call(mu.T, random_table, items_tc)
    lut = lut2d.reshape(-1)
    bnd = bnd2d[0]

    weighted, total_ll = _sc_call(lut, bnd, anno_p, annot_p, items_p, conf_p)
    return (weighted[:, :N], total_ll[:, :NUM_ITEMS])
